# baseline (speedup 1.0000x reference)
import jax
import jax.numpy as jnp
from jax.experimental import pallas as pl

_B, _S, _H, _D = 4, 4096, 1024, 6
_TB = 512


def _moe_block(x_ref, m_ref, bias_ref, noise_ref, pred_ref, assign_ref, probs_ref):
    x = x_ref[...]
    r = jnp.dot(x, m_ref[...], preferred_element_type=jnp.float32)
    r = r + bias_ref[...]

    logits = r[:, 0:6]
    mean = r[:, 6:7]
    s4 = r[:, 7:8]
    s6 = r[:, 8:9]
    s8 = r[:, 9:10]
    s610 = r[:, 10:11]

    sumsq = jnp.sum(x * x, axis=1, keepdims=True)
    var = (sumsq - _H * mean * mean) / (_H - 1)
    std = jnp.sqrt(jnp.maximum(var, 0.0))

    mx = jnp.max(logits, axis=1, keepdims=True)
    ex = jnp.exp(logits - mx)
    probs = ex / jnp.sum(ex, axis=1, keepdims=True)
    assign = jnp.argmax(probs, axis=1).astype(jnp.int32)[:, None]

    sig_mean = jax.nn.sigmoid(mean)
    p0 = jnp.tanh(s4) * (1.0 + std)
    p1 = sig_mean * 0.3 - 0.15
    p2 = s6 * 0.8 + jnp.sin(s610 * 3.14159) * 0.4
    p3 = jnp.tanh(s8) * 0.9 + noise_ref[...]
    rm = jnp.maximum(mean, 0.0)
    p4 = jnp.where(rm > 0.0,
                   jnp.exp(1.2 * jnp.log(jnp.maximum(rm, 1e-38))),
                   0.0) + std * 2.5 - 0.5
    p5 = sig_mean * 0.4 + jnp.tanh(std) * 0.2

    pred = ((assign == 0).astype(jnp.float32) * p0 * probs[:, 0:1]
            + (assign == 1).astype(jnp.float32) * p1 * probs[:, 1:2]
            + (assign == 2).astype(jnp.float32) * p2 * probs[:, 2:3]
            + (assign == 3).astype(jnp.float32) * p3 * probs[:, 3:4]
            + (assign == 4).astype(jnp.float32) * p4 * probs[:, 4:5]
            + (assign == 5).astype(jnp.float32) * p5 * probs[:, 5:6])

    pred_ref[...] = pred
    assign_ref[...] = assign
    probs_ref[...] = probs


def kernel(sequence_embeddings, market_volatility, risk_factors, router_weight, router_bias):
    del market_volatility, risk_factors
    bs = _B * _S
    x = sequence_embeddings.reshape(bs, _H)

    idx = jnp.arange(_H, dtype=jnp.float32)[:, None]
    cols = [
        router_weight.T,
        jnp.full((_H, 1), 1.0 / _H, dtype=jnp.float32),
        (idx < 4).astype(jnp.float32) / 4.0,
        (idx < 6).astype(jnp.float32) / 6.0,
        (idx < 8).astype(jnp.float32) / 8.0,
        ((idx >= 6) & (idx < 10)).astype(jnp.float32) / 4.0,
        jnp.zeros((_H, 5), dtype=jnp.float32),
    ]
    mred = jnp.concatenate(cols, axis=1)
    bias_pad = jnp.concatenate(
        [router_bias, jnp.zeros((10,), dtype=jnp.float32)]).reshape(1, 16)
    noise = (jax.random.normal(jax.random.key(1234), (_B, _S, 1),
                               dtype=jnp.float32) * 0.05).reshape(bs, 1)

    grid = (bs // _TB,)
    pred, assign, probs = pl.pallas_call(
        _moe_block,
        grid=grid,
        in_specs=[
            pl.BlockSpec((_TB, _H), lambda i: (i, 0)),
            pl.BlockSpec((_H, 16), lambda i: (0, 0)),
            pl.BlockSpec((1, 16), lambda i: (0, 0)),
            pl.BlockSpec((_TB, 1), lambda i: (i, 0)),
        ],
        out_specs=[
            pl.BlockSpec((_TB, 1), lambda i: (i, 0)),
            pl.BlockSpec((_TB, 1), lambda i: (i, 0)),
            pl.BlockSpec((_TB, 6), lambda i: (i, 0)),
        ],
        out_shape=[
            jax.ShapeDtypeStruct((bs, 1), jnp.float32),
            jax.ShapeDtypeStruct((bs, 1), jnp.int32),
            jax.ShapeDtypeStruct((bs, 6), jnp.float32),
        ],
    )(x, mred, bias_pad, noise)

    return (pred.reshape(_B, _S, 1),
            assign.reshape(_B, _S),
            probs.reshape(_B, _S, _D))


# speedup vs baseline: 3.6433x; 3.6433x over previous
import jax
import jax.numpy as jnp
from jax import lax
from jax.experimental import pallas as pl

_B, _S, _H, _D = 4, 4096, 1024, 6
_TB = 512


def _moe_block(x_ref, m_ref, ones_ref, bias_ref, noise_ref,
               pred_ref, assign_ref, probs_ref):
    x = x_ref[...]
    xb = x.astype(jnp.bfloat16)
    r = lax.dot_general(m_ref[...], xb, (((1,), (1,)), ((), ())),
                        preferred_element_type=jnp.float32)
    r = r + bias_ref[...]

    logits = r[0:6, :]
    mean = r[6:7, :]
    s4 = r[7:8, :]
    s6 = r[8:9, :]
    s8 = r[9:10, :]
    s610 = r[10:11, :]

    xsq = (xb * xb).astype(jnp.bfloat16)
    sumsq = lax.dot_general(ones_ref[...], xsq, (((1,), (1,)), ((), ())),
                            preferred_element_type=jnp.float32)
    var = (sumsq - _H * mean * mean) / (_H - 1)
    std = jnp.sqrt(jnp.maximum(var, 0.0))

    mx = jnp.max(logits, axis=0, keepdims=True)
    ex = jnp.exp(logits - mx)
    probs = ex / jnp.sum(ex, axis=0, keepdims=True)
    assign = jnp.argmax(probs, axis=0).astype(jnp.int32)[None, :]

    sig_mean = jax.nn.sigmoid(mean)
    p0 = jnp.tanh(s4) * (1.0 + std)
    p1 = sig_mean * 0.3 - 0.15
    p2 = s6 * 0.8 + jnp.sin(s610 * 3.14159) * 0.4
    p3 = jnp.tanh(s8) * 0.9 + noise_ref[0]
    rm = jnp.maximum(mean, 0.0)
    p4 = jnp.where(rm > 0.0,
                   jnp.exp(1.2 * jnp.log(jnp.maximum(rm, 1e-38))),
                   0.0) + std * 2.5 - 0.5
    p5 = sig_mean * 0.4 + jnp.tanh(std) * 0.2

    pred = ((assign == 0).astype(jnp.float32) * p0 * probs[0:1, :]
            + (assign == 1).astype(jnp.float32) * p1 * probs[1:2, :]
            + (assign == 2).astype(jnp.float32) * p2 * probs[2:3, :]
            + (assign == 3).astype(jnp.float32) * p3 * probs[3:4, :]
            + (assign == 4).astype(jnp.float32) * p4 * probs[4:5, :]
            + (assign == 5).astype(jnp.float32) * p5 * probs[5:6, :])

    pred_ref[0] = pred
    assign_ref[0] = assign
    probs_ref[...] = probs


def kernel(sequence_embeddings, market_volatility, risk_factors, router_weight, router_bias):
    del market_volatility, risk_factors
    bs = _B * _S
    nblk = bs // _TB
    x = sequence_embeddings.reshape(bs, _H)

    idx = jnp.arange(_H, dtype=jnp.float32)[:, None]
    cols = [
        router_weight.T,
        jnp.full((_H, 1), 1.0 / _H, dtype=jnp.float32),
        (idx < 4).astype(jnp.float32) / 4.0,
        (idx < 6).astype(jnp.float32) / 6.0,
        (idx < 8).astype(jnp.float32) / 8.0,
        ((idx >= 6) & (idx < 10)).astype(jnp.float32) / 4.0,
        jnp.zeros((_H, 5), dtype=jnp.float32),
    ]
    mred = jnp.concatenate(cols, axis=1).T.astype(jnp.bfloat16)
    ones_row = jnp.ones((1, _H), dtype=jnp.bfloat16)
    bias_col = jnp.concatenate(
        [router_bias, jnp.zeros((10,), dtype=jnp.float32)]).reshape(16, 1)
    noise = (jax.random.normal(jax.random.key(1234), (_B, _S, 1),
                               dtype=jnp.float32) * 0.05).reshape(nblk, 1, _TB)

    grid = (nblk,)
    pred, assign, probs = pl.pallas_call(
        _moe_block,
        grid=grid,
        in_specs=[
            pl.BlockSpec((_TB, _H), lambda i: (i, 0)),
            pl.BlockSpec((16, _H), lambda i: (0, 0)),
            pl.BlockSpec((1, _H), lambda i: (0, 0)),
            pl.BlockSpec((16, 1), lambda i: (0, 0)),
            pl.BlockSpec((1, 1, _TB), lambda i: (i, 0, 0)),
        ],
        out_specs=[
            pl.BlockSpec((1, 1, _TB), lambda i: (i, 0, 0)),
            pl.BlockSpec((1, 1, _TB), lambda i: (i, 0, 0)),
            pl.BlockSpec((6, _TB), lambda i: (0, i)),
        ],
        out_shape=[
            jax.ShapeDtypeStruct((nblk, 1, _TB), jnp.float32),
            jax.ShapeDtypeStruct((nblk, 1, _TB), jnp.int32),
            jax.ShapeDtypeStruct((6, bs), jnp.float32),
        ],
    )(x, mred, ones_row, bias_col, noise)

    return (pred.reshape(_B, _S, 1),
            assign.reshape(_B, _S),
            probs.T.reshape(_B, _S, _D))


# speedup vs baseline: 4.3734x; 1.2004x over previous
import jax
import jax.numpy as jnp
from jax import lax
from jax.experimental import pallas as pl

_B, _S, _H, _D = 4, 4096, 1024, 6
_TB = 1024


def _moe_block(x_ref, m_ref, ones_ref, bias_ref, noise_ref,
               pred_ref, assign_ref, probs_ref):
    x = x_ref[...]
    xb = x.astype(jnp.bfloat16)
    r = lax.dot_general(m_ref[...], xb, (((1,), (1,)), ((), ())),
                        preferred_element_type=jnp.float32)
    r = r + bias_ref[...]

    logits = r[0:6, :]
    mean = r[6:7, :]
    s4 = r[7:8, :]
    s6 = r[8:9, :]
    s8 = r[9:10, :]
    s610 = r[10:11, :]

    xsq = (xb * xb).astype(jnp.bfloat16)
    sumsq = lax.dot_general(ones_ref[...], xsq, (((1,), (1,)), ((), ())),
                            preferred_element_type=jnp.float32)
    var = (sumsq - _H * mean * mean) / (_H - 1)
    std = jnp.sqrt(jnp.maximum(var, 0.0))

    mx = jnp.max(logits, axis=0, keepdims=True)
    ex = jnp.exp(logits - mx)
    probs = ex / jnp.sum(ex, axis=0, keepdims=True)
    assign = jnp.argmax(probs, axis=0).astype(jnp.int32)[None, :]

    sig_mean = jax.nn.sigmoid(mean)
    p0 = jnp.tanh(s4) * (1.0 + std)
    p1 = sig_mean * 0.3 - 0.15
    p2 = s6 * 0.8 + jnp.sin(s610 * 3.14159) * 0.4
    p3 = jnp.tanh(s8) * 0.9 + noise_ref[0]
    rm = jnp.maximum(mean, 0.0)
    p4 = jnp.where(rm > 0.0,
                   jnp.exp(1.2 * jnp.log(jnp.maximum(rm, 1e-38))),
                   0.0) + std * 2.5 - 0.5
    p5 = sig_mean * 0.4 + jnp.tanh(std) * 0.2

    pred = ((assign == 0).astype(jnp.float32) * p0 * probs[0:1, :]
            + (assign == 1).astype(jnp.float32) * p1 * probs[1:2, :]
            + (assign == 2).astype(jnp.float32) * p2 * probs[2:3, :]
            + (assign == 3).astype(jnp.float32) * p3 * probs[3:4, :]
            + (assign == 4).astype(jnp.float32) * p4 * probs[4:5, :]
            + (assign == 5).astype(jnp.float32) * p5 * probs[5:6, :])

    pred_ref[0] = pred
    assign_ref[0] = assign
    probs_ref[...] = probs


def kernel(sequence_embeddings, market_volatility, risk_factors, router_weight, router_bias):
    del market_volatility, risk_factors
    bs = _B * _S
    nblk = bs // _TB
    x = sequence_embeddings.reshape(bs, _H)

    idx = jnp.arange(_H, dtype=jnp.float32)[:, None]
    cols = [
        router_weight.T,
        jnp.full((_H, 1), 1.0 / _H, dtype=jnp.float32),
        (idx < 4).astype(jnp.float32) / 4.0,
        (idx < 6).astype(jnp.float32) / 6.0,
        (idx < 8).astype(jnp.float32) / 8.0,
        ((idx >= 6) & (idx < 10)).astype(jnp.float32) / 4.0,
        jnp.zeros((_H, 5), dtype=jnp.float32),
    ]
    mred = jnp.concatenate(cols, axis=1).T.astype(jnp.bfloat16)
    ones_row = jnp.ones((1, _H), dtype=jnp.bfloat16)
    bias_col = jnp.concatenate(
        [router_bias, jnp.zeros((10,), dtype=jnp.float32)]).reshape(16, 1)
    noise = (jax.random.normal(jax.random.key(1234), (_B, _S, 1),
                               dtype=jnp.float32) * 0.05).reshape(nblk, 1, _TB)

    grid = (nblk,)
    pred, assign, probs = pl.pallas_call(
        _moe_block,
        grid=grid,
        in_specs=[
            pl.BlockSpec((_TB, _H), lambda i: (i, 0)),
            pl.BlockSpec((16, _H), lambda i: (0, 0)),
            pl.BlockSpec((1, _H), lambda i: (0, 0)),
            pl.BlockSpec((16, 1), lambda i: (0, 0)),
            pl.BlockSpec((1, 1, _TB), lambda i: (i, 0, 0)),
        ],
        out_specs=[
            pl.BlockSpec((1, 1, _TB), lambda i: (i, 0, 0)),
            pl.BlockSpec((1, 1, _TB), lambda i: (i, 0, 0)),
            pl.BlockSpec((6, _TB), lambda i: (0, i)),
        ],
        out_shape=[
            jax.ShapeDtypeStruct((nblk, 1, _TB), jnp.float32),
            jax.ShapeDtypeStruct((nblk, 1, _TB), jnp.int32),
            jax.ShapeDtypeStruct((6, bs), jnp.float32),
        ],
    )(x, mred, ones_row, bias_col, noise)

    return (pred.reshape(_B, _S, 1),
            assign.reshape(_B, _S),
            probs.T.reshape(_B, _S, _D))


# speedup vs baseline: 4.6222x; 1.0569x over previous
import jax
import jax.numpy as jnp
from jax import lax
from jax.experimental import pallas as pl

_B, _S, _H, _D = 4, 4096, 1024, 6
_TB = 2048


def _moe_block(x_ref, m_ref, ones_ref, bias_ref, noise_ref,
               pred_ref, assign_ref, probs_ref):
    x = x_ref[...]
    xb = x.astype(jnp.bfloat16)
    r = lax.dot_general(m_ref[...], xb, (((1,), (1,)), ((), ())),
                        preferred_element_type=jnp.float32)
    r = r + bias_ref[...]

    logits = r[0:6, :]
    mean = r[6:7, :]
    s4 = r[7:8, :]
    s6 = r[8:9, :]
    s8 = r[9:10, :]
    s610 = r[10:11, :]

    xsq = (xb * xb).astype(jnp.bfloat16)
    sumsq = lax.dot_general(ones_ref[...], xsq, (((1,), (1,)), ((), ())),
                            preferred_element_type=jnp.float32)
    var = (sumsq - _H * mean * mean) / (_H - 1)
    std = jnp.sqrt(jnp.maximum(var, 0.0))

    mx = jnp.max(logits, axis=0, keepdims=True)
    ex = jnp.exp(logits - mx)
    probs = ex / jnp.sum(ex, axis=0, keepdims=True)
    assign = jnp.argmax(probs, axis=0).astype(jnp.int32)[None, :]

    sig_mean = jax.nn.sigmoid(mean)
    p0 = jnp.tanh(s4) * (1.0 + std)
    p1 = sig_mean * 0.3 - 0.15
    p2 = s6 * 0.8 + jnp.sin(s610 * 3.14159) * 0.4
    p3 = jnp.tanh(s8) * 0.9 + noise_ref[0]
    rm = jnp.maximum(mean, 0.0)
    p4 = jnp.where(rm > 0.0,
                   jnp.exp(1.2 * jnp.log(jnp.maximum(rm, 1e-38))),
                   0.0) + std * 2.5 - 0.5
    p5 = sig_mean * 0.4 + jnp.tanh(std) * 0.2

    pred = ((assign == 0).astype(jnp.float32) * p0 * probs[0:1, :]
            + (assign == 1).astype(jnp.float32) * p1 * probs[1:2, :]
            + (assign == 2).astype(jnp.float32) * p2 * probs[2:3, :]
            + (assign == 3).astype(jnp.float32) * p3 * probs[3:4, :]
            + (assign == 4).astype(jnp.float32) * p4 * probs[4:5, :]
            + (assign == 5).astype(jnp.float32) * p5 * probs[5:6, :])

    pred_ref[0] = pred
    assign_ref[0] = assign
    probs_ref[...] = probs


def kernel(sequence_embeddings, market_volatility, risk_factors, router_weight, router_bias):
    del market_volatility, risk_factors
    bs = _B * _S
    nblk = bs // _TB
    x = sequence_embeddings.reshape(bs, _H)

    idx = jnp.arange(_H, dtype=jnp.float32)[:, None]
    cols = [
        router_weight.T,
        jnp.full((_H, 1), 1.0 / _H, dtype=jnp.float32),
        (idx < 4).astype(jnp.float32) / 4.0,
        (idx < 6).astype(jnp.float32) / 6.0,
        (idx < 8).astype(jnp.float32) / 8.0,
        ((idx >= 6) & (idx < 10)).astype(jnp.float32) / 4.0,
        jnp.zeros((_H, 5), dtype=jnp.float32),
    ]
    mred = jnp.concatenate(cols, axis=1).T.astype(jnp.bfloat16)
    ones_row = jnp.ones((1, _H), dtype=jnp.bfloat16)
    bias_col = jnp.concatenate(
        [router_bias, jnp.zeros((10,), dtype=jnp.float32)]).reshape(16, 1)
    noise = (jax.random.normal(jax.random.key(1234), (_B, _S, 1),
                               dtype=jnp.float32) * 0.05).reshape(nblk, 1, _TB)

    grid = (nblk,)
    pred, assign, probs = pl.pallas_call(
        _moe_block,
        grid=grid,
        in_specs=[
            pl.BlockSpec((_TB, _H), lambda i: (i, 0)),
            pl.BlockSpec((16, _H), lambda i: (0, 0)),
            pl.BlockSpec((1, _H), lambda i: (0, 0)),
            pl.BlockSpec((16, 1), lambda i: (0, 0)),
            pl.BlockSpec((1, 1, _TB), lambda i: (i, 0, 0)),
        ],
        out_specs=[
            pl.BlockSpec((1, 1, _TB), lambda i: (i, 0, 0)),
            pl.BlockSpec((1, 1, _TB), lambda i: (i, 0, 0)),
            pl.BlockSpec((6, _TB), lambda i: (0, i)),
        ],
        out_shape=[
            jax.ShapeDtypeStruct((nblk, 1, _TB), jnp.float32),
            jax.ShapeDtypeStruct((nblk, 1, _TB), jnp.int32),
            jax.ShapeDtypeStruct((6, bs), jnp.float32),
        ],
    )(x, mred, ones_row, bias_col, noise)

    return (pred.reshape(_B, _S, 1),
            assign.reshape(_B, _S),
            probs.T.reshape(_B, _S, _D))
